# baseline (speedup 1.0000x reference)
import functools

import jax
import jax.numpy as jnp
from jax import lax
from jax.experimental import pallas as pl
from jax.experimental.pallas import tpu as pltpu
from jax.experimental.pallas import tpu_sc as plsc

_NW = 32
_CHUNK = 40


def kernel(tokens, lookup_table):
    B, S = tokens.shape
    V, D = lookup_table.shape
    N = B * S
    idx = tokens.reshape(N).astype(jnp.int32)
    n_per_w = N // _NW
    n_chunks = n_per_w // _CHUNK

    mesh = plsc.VectorSubcoreMesh(core_axis_name="c", subcore_axis_name="s")

    @functools.partial(
        pl.kernel,
        mesh=mesh,
        compiler_params=pltpu.CompilerParams(use_tc_tiling_on_sc=False),
        out_type=jax.ShapeDtypeStruct((N, D), jnp.float32),
        scratch_types=[
            pltpu.VMEM((n_per_w,), jnp.int32),
            pltpu.VMEM((_CHUNK, D), jnp.float32),
            pltpu.SemaphoreType.DMA,
        ],
    )
    def _gather(table_hbm, idx_hbm, out_hbm, idx_v, buf, sem):
        wid = lax.axis_index("s") * 2 + lax.axis_index("c")
        base = wid * n_per_w
        pltpu.sync_copy(idx_hbm.at[pl.ds(base, n_per_w)], idx_v)

        def body(c, carry):
            off = c * _CHUNK
            pltpu.async_copy(
                table_hbm.at[idx_v.at[pl.ds(off, _CHUNK)]], buf, sem
            ).wait()
            pltpu.sync_copy(buf, out_hbm.at[pl.ds(base + off, _CHUNK)])
            return carry

        lax.fori_loop(0, n_chunks, body, 0)

    out = _gather(lookup_table, idx)
    return out.reshape(B, S, D)


# speedup vs baseline: 1.0360x; 1.0360x over previous
import functools

import jax
import jax.numpy as jnp
from jax import lax
from jax.experimental import pallas as pl
from jax.experimental.pallas import tpu as pltpu
from jax.experimental.pallas import tpu_sc as plsc

_NW = 32
_CHUNK = 64
_NBUF = 2


def kernel(tokens, lookup_table):
    B, S = tokens.shape
    V, D = lookup_table.shape
    N = B * S
    idx = tokens.reshape(N).astype(jnp.int32)
    n_per_w = N // _NW
    n_chunks = n_per_w // _CHUNK

    mesh = plsc.VectorSubcoreMesh(core_axis_name="c", subcore_axis_name="s")

    @functools.partial(
        pl.kernel,
        mesh=mesh,
        compiler_params=pltpu.CompilerParams(use_tc_tiling_on_sc=False),
        out_type=jax.ShapeDtypeStruct((N, D), jnp.float32),
        scratch_types=[
            pltpu.VMEM((n_per_w,), jnp.int32),
            pltpu.VMEM((_CHUNK, D), jnp.float32),
            pltpu.VMEM((_CHUNK, D), jnp.float32),
            pltpu.SemaphoreType.DMA,
            pltpu.SemaphoreType.DMA,
            pltpu.SemaphoreType.DMA,
            pltpu.SemaphoreType.DMA,
        ],
    )
    def _gather(table_hbm, idx_hbm, out_hbm, idx_v, buf0, buf1,
                gsem0, gsem1, ssem0, ssem1):
        wid = lax.axis_index("s") * 2 + lax.axis_index("c")
        base = wid * n_per_w
        pltpu.sync_copy(idx_hbm.at[pl.ds(base, n_per_w)], idx_v)

        bufs = (buf0, buf1)
        gsems = (gsem0, gsem1)
        ssems = (ssem0, ssem1)

        def start_gather(cc, b):
            pltpu.async_copy(
                table_hbm.at[idx_v.at[pl.ds(cc * _CHUNK, _CHUNK)]],
                bufs[b], gsems[b])

        def start_scatter(cc, b):
            pltpu.async_copy(
                bufs[b], out_hbm.at[pl.ds(base + cc * _CHUNK, _CHUNK)],
                ssems[b])

        def wait_gather(b):
            pltpu.make_async_copy(
                table_hbm.at[pl.ds(0, _CHUNK)], bufs[b], gsems[b]).wait()

        def wait_scatter(b):
            pltpu.make_async_copy(
                bufs[b], out_hbm.at[pl.ds(base, _CHUNK)], ssems[b]).wait()

        for b in range(_NBUF):
            start_gather(b, b)

        def body(g, carry):
            c = g * _NBUF
            for b in range(_NBUF):
                cc = c + b

                @pl.when(cc < n_chunks)
                def _():
                    wait_gather(b)
                    start_scatter(cc, b)
                    wait_scatter(b)

                    @pl.when(cc + _NBUF < n_chunks)
                    def _():
                        start_gather(cc + _NBUF, b)

            return carry

        lax.fori_loop(0, (n_chunks + _NBUF - 1) // _NBUF, body, 0)

    out = _gather(lookup_table, idx)
    return out.reshape(B, S, D)


# speedup vs baseline: 1.0396x; 1.0035x over previous
import functools

import jax
import jax.numpy as jnp
from jax import lax
from jax.experimental import pallas as pl
from jax.experimental.pallas import tpu as pltpu
from jax.experimental.pallas import tpu_sc as plsc

_NW = 32
_NBUF = 2


def kernel(tokens, lookup_table):
    B, S = tokens.shape
    V, D = lookup_table.shape
    idx = tokens.astype(jnp.int32)
    b_per_w = B // _NW

    mesh = plsc.VectorSubcoreMesh(core_axis_name="c", subcore_axis_name="s")

    @functools.partial(
        pl.kernel,
        mesh=mesh,
        compiler_params=pltpu.CompilerParams(use_tc_tiling_on_sc=False),
        out_type=jax.ShapeDtypeStruct((B, S, D), jnp.float32),
        scratch_types=[
            pltpu.VMEM((b_per_w, S), jnp.int32),
            pltpu.VMEM((S, D), jnp.float32),
            pltpu.VMEM((S, D), jnp.float32),
            pltpu.SemaphoreType.DMA,
            pltpu.SemaphoreType.DMA,
            pltpu.SemaphoreType.DMA,
            pltpu.SemaphoreType.DMA,
        ],
    )
    def _gather(table_hbm, idx_hbm, out_hbm, idx_v, buf0, buf1,
                gsem0, gsem1, ssem0, ssem1):
        wid = lax.axis_index("s") * 2 + lax.axis_index("c")
        base = wid * b_per_w
        pltpu.sync_copy(idx_hbm.at[pl.ds(base, b_per_w)], idx_v)

        bufs = (buf0, buf1)
        gsems = (gsem0, gsem1)
        ssems = (ssem0, ssem1)

        def start_gather(bl, p):
            pltpu.async_copy(
                table_hbm.at[idx_v.at[bl]], bufs[p], gsems[p])

        def start_scatter(bl, p):
            pltpu.async_copy(bufs[p], out_hbm.at[base + bl], ssems[p])

        def wait_gather(p):
            pltpu.make_async_copy(
                table_hbm.at[idx_v.at[0]], bufs[p], gsems[p]).wait()

        def wait_scatter(p):
            pltpu.make_async_copy(
                bufs[p], out_hbm.at[base], ssems[p]).wait()

        for p in range(_NBUF):
            start_gather(p, p)

        def body(g, carry):
            bl = g * _NBUF
            for p in range(_NBUF):
                blp = bl + p

                @pl.when(blp < b_per_w)
                def _():
                    wait_gather(p)
                    start_scatter(blp, p)
                    wait_scatter(p)

                    @pl.when(blp + _NBUF < b_per_w)
                    def _():
                        start_gather(blp + _NBUF, p)

            return carry

        lax.fori_loop(0, (b_per_w + _NBUF - 1) // _NBUF, body, 0)

    return _gather(lookup_table, idx)


# speedup vs baseline: 1.4309x; 1.3764x over previous
import functools

import jax
import jax.numpy as jnp
from jax import lax
from jax.experimental import pallas as pl
from jax.experimental.pallas import tpu as pltpu
from jax.experimental.pallas import tpu_sc as plsc

_NW = 32
_NBUF = 2


def kernel(tokens, lookup_table):
    B, S = tokens.shape
    V, D = lookup_table.shape
    Dp = 1024
    SL = Dp // 128
    b_per_w = B // _NW
    idx3 = tokens.astype(jnp.int32).reshape(_NW, b_per_w, S)
    table3 = jnp.pad(lookup_table, ((0, 0), (0, Dp - D))).reshape(V, SL, 128)

    mesh = plsc.VectorSubcoreMesh(core_axis_name="c", subcore_axis_name="s")

    @functools.partial(
        pl.kernel,
        mesh=mesh,
        out_type=jax.ShapeDtypeStruct((B, S, SL, 128), jnp.float32),
        scratch_types=[
            pltpu.VMEM((b_per_w, S), jnp.int32),
            pltpu.VMEM((S, SL, 128), jnp.float32),
            pltpu.VMEM((S, SL, 128), jnp.float32),
            pltpu.SemaphoreType.DMA,
            pltpu.SemaphoreType.DMA,
            pltpu.SemaphoreType.DMA,
            pltpu.SemaphoreType.DMA,
        ],
    )
    def _gather(table_hbm, idx_hbm, out_hbm, idx_v, buf0, buf1,
                gsem0, gsem1, ssem0, ssem1):
        wid = lax.axis_index("s") * 2 + lax.axis_index("c")
        base = wid * b_per_w
        pltpu.sync_copy(idx_hbm.at[wid], idx_v)

        bufs = (buf0, buf1)
        gsems = (gsem0, gsem1)
        ssems = (ssem0, ssem1)

        def start_gather(bl, p):
            pltpu.async_copy(
                table_hbm.at[idx_v.at[bl]], bufs[p], gsems[p])

        def start_scatter(bl, p):
            pltpu.async_copy(bufs[p], out_hbm.at[base + bl], ssems[p])

        def wait_gather(p):
            pltpu.make_async_copy(
                table_hbm.at[idx_v.at[0]], bufs[p], gsems[p]).wait()

        def wait_scatter(p):
            pltpu.make_async_copy(
                bufs[p], out_hbm.at[base], ssems[p]).wait()

        for p in range(_NBUF):
            start_gather(p, p)

        def body(g, carry):
            bl = g * _NBUF
            for p in range(_NBUF):
                blp = bl + p

                @pl.when(blp < b_per_w)
                def _():
                    wait_gather(p)
                    start_scatter(blp, p)
                    wait_scatter(p)

                    @pl.when(blp + _NBUF < b_per_w)
                    def _():
                        start_gather(blp + _NBUF, p)

            return carry

        lax.fori_loop(0, (b_per_w + _NBUF - 1) // _NBUF, body, 0)

    out4 = _gather(table3, idx3)
    return out4.reshape(B, S, Dp)[:, :, :D]
